# baseline (speedup 1.0000x reference)
import functools

import jax
import jax.numpy as jnp
from jax import lax
from jax.experimental import pallas as pl
from jax.experimental.pallas import tpu as pltpu
from jax.experimental.pallas import tpu_sc as plsc

B, S, L = 1024, 20, 20
VOCAB, D, NCLASS = 100000, 128, 100

NIDX = S * L
CHUNK = 80
CPB = NIDX // CHUNK
NW = 32
BPW = B // NW
NVREG = D // 16
UNROLL = 4
NGRP = S // 4


def _sc_gather_maxpool(x, table):
    mesh = plsc.VectorSubcoreMesh(core_axis_name="c", subcore_axis_name="s")

    @functools.partial(
        pl.kernel,
        mesh=mesh,
        out_type=jax.ShapeDtypeStruct((B, D), jnp.float32),
        scratch_types=[
            pltpu.VMEM((S, L), jnp.int32),
            pltpu.VMEM((S, L), jnp.int32),
            pltpu.VMEM((NIDX,), jnp.int32),
            pltpu.VMEM((NIDX,), jnp.int32),
        ] + [pltpu.VMEM((CHUNK, D), jnp.float32)] * 10 + [
            pltpu.VMEM((BPW, D), jnp.float32),
            pltpu.VMEM((2, 32), jnp.int32),
        ] + [pltpu.SemaphoreType.DMA] * 12,
    )
    def k(x_hbm, table_hbm, out_hbm, ib0, ib1, il0, il1, *rest):
        rows = rest[0:10]
        out_v, rs = rest[10], rest[11]
        isems = rest[12:14]
        sems = rest[14:24]
        wid = lax.axis_index("s") * 2 + lax.axis_index("c")
        base = wid * BPW

        ibs = (ib0, ib1)
        ils = (il0, il1)


        def stage(b, i):
            pltpu.async_copy(x_hbm.at[base + b], ibs[i], isems[i])

        def stage_wait(i):
            pltpu.make_async_copy(x_hbm.at[0], ibs[i], isems[i]).wait()

        def flatten(i):
            for t in range(NGRP):
                ils[i][pl.ds(80 * t, 16)] = ibs[i][4 * t, pl.ds(0, 16)]
                for v, kk in ((1, 12), (2, 8), (3, 4)):
                    rr = (t * 3 + v) % 2
                    rs[rr, pl.ds(0, 16)] = ibs[i][4 * t + v - 1, pl.ds(4, 16)]
                    rs[rr, pl.ds(16, 16)] = ibs[i][4 * t + v, pl.ds(0, 16)]
                    ils[i][pl.ds(80 * t + 16 * v, 16)] = rs[rr, pl.ds(kk, 16)]
                ils[i][pl.ds(80 * t + 64, 16)] = ibs[i][4 * t + 3, pl.ds(4, 16)]

        def fire(i, kk, slot):
            pltpu.async_copy(
                table_hbm.at[ils[i].at[pl.ds(kk * CHUNK, CHUNK)]],
                rows[slot], sems[slot],
            )

        def drain(slot):
            pltpu.make_async_copy(
                table_hbm.at[ils[0].at[pl.ds(0, CHUNK)]], rows[slot], sems[slot]
            ).wait()

        def reduce_chunk(rref, acc):
            def body(r, acc):
                for u in range(UNROLL):
                    acc = tuple(
                        jnp.maximum(acc[j], rref[r * UNROLL + u, pl.ds(j * 16, 16)])
                        for j in range(NVREG)
                    )
                return acc
            return lax.fori_loop(0, CHUNK // UNROLL, body, acc)

        stage(0, 0)
        stage_wait(0)
        flatten(0)
        for kk in range(CPB):
            fire(0, kk, kk)
        stage(1, 1)
        stage_wait(1)
        flatten(1)
        stage(2, 0)

        def batch_body(b, i, ni, mybase, nbase):
            for kk in range(CPB):
                @pl.when(b + 1 < BPW)
                def _():
                    fire(ni, kk, nbase + kk)
            acc = tuple(
                jnp.full((16,), -jnp.inf, jnp.float32) for _ in range(NVREG)
            )
            for kk in range(CPB):
                drain(mybase + kk)
                acc = reduce_chunk(rows[mybase + kk], acc)
            for j in range(NVREG):
                out_v[b, pl.ds(j * 16, 16)] = acc[j]
            @pl.when(b + 3 < BPW)
            def _():
                stage(b + 3, ni)
            @pl.when(b + 2 < BPW)
            def _():
                stage_wait(i)
            flatten(i)

        def pair_body(p, _):
            batch_body(2 * p, 0, 1, 0, CPB)
            batch_body(2 * p + 1, 1, 0, CPB, 0)
            return 0

        lax.fori_loop(0, BPW // 2, pair_body, 0)
        pltpu.sync_copy(out_v, out_hbm.at[pl.ds(base, BPW)])

    return k(x, table)


def _fc_sigmoid(h, W, b2):
    def fc_kernel(h_ref, w_ref, b_ref, o_ref):
        acc = lax.dot_general(
            h_ref[...], w_ref[...],
            dimension_numbers=(((1,), (1,)), ((), ())),
            preferred_element_type=jnp.float32,
        )
        o_ref[...] = jax.nn.sigmoid(acc + b_ref[...])

    return pl.pallas_call(
        fc_kernel,
        out_shape=jax.ShapeDtypeStruct((B, NCLASS), jnp.float32),
    )(h, W, b2)


def kernel(x, table, W, b):
    h = _sc_gather_maxpool(x.astype(jnp.int32), table)
    return _fc_sigmoid(h, W, b.reshape(1, NCLASS))


# speedup vs baseline: 1.1068x; 1.1068x over previous
import functools

import jax
import jax.numpy as jnp
from jax import lax
from jax.experimental import pallas as pl
from jax.experimental.pallas import tpu as pltpu
from jax.experimental.pallas import tpu_sc as plsc

B, S, L = 1024, 20, 20
VOCAB, D, NCLASS = 100000, 128, 100

NIDX = S * L
CHUNK = 80
CPB = NIDX // CHUNK
NW = 32
BPW = B // NW
NVREG = D // 16
UNROLL = 4
NGRP = S // 4


def _sc_gather_maxpool(x, table):
    mesh = plsc.VectorSubcoreMesh(core_axis_name="c", subcore_axis_name="s")

    @functools.partial(
        pl.kernel,
        mesh=mesh,
        out_type=jax.ShapeDtypeStruct((B, D), jnp.float32),
        scratch_types=[
            pltpu.VMEM((S, L), jnp.int32),
            pltpu.VMEM((S, L), jnp.int32),
            pltpu.VMEM((NIDX,), jnp.int32),
            pltpu.VMEM((NIDX,), jnp.int32),
            pltpu.VMEM((CHUNK, D), jnp.float32),
            pltpu.VMEM((CHUNK, D), jnp.float32),
            pltpu.VMEM((CHUNK, D), jnp.float32),
            pltpu.VMEM((CHUNK, D), jnp.float32),
            pltpu.VMEM((CHUNK, D), jnp.float32),
            pltpu.VMEM((BPW, D), jnp.float32),
            pltpu.VMEM((1, 32), jnp.int32),
            pltpu.SemaphoreType.DMA,
            pltpu.SemaphoreType.DMA,
            pltpu.SemaphoreType.DMA,
            pltpu.SemaphoreType.DMA,
            pltpu.SemaphoreType.DMA,
            pltpu.SemaphoreType.DMA,
            pltpu.SemaphoreType.DMA,
        ],
    )
    def k(x_hbm, table_hbm, out_hbm, ib0, ib1, il0, il1, rows0, rows1,
          rows2, rows3, rows4, out_v, rs, isem0, isem1, sem0, sem1, sem2,
          sem3, sem4):
        wid = lax.axis_index("s") * 2 + lax.axis_index("c")
        base = wid * BPW

        ibs = (ib0, ib1)
        ils = (il0, il1)
        isems = (isem0, isem1)
        rows = (rows0, rows1, rows2, rows3, rows4)
        sems = (sem0, sem1, sem2, sem3, sem4)


        def stage(b, i):
            pltpu.async_copy(x_hbm.at[base + b], ibs[i], isems[i])

        def stage_wait(i):
            pltpu.make_async_copy(x_hbm.at[0], ibs[i], isems[i]).wait()

        def flatten(i):
            for t in range(NGRP):
                ils[i][pl.ds(80 * t, 16)] = ibs[i][4 * t, pl.ds(0, 16)]
                for v, kk in ((1, 12), (2, 8), (3, 4)):
                    rs[0, pl.ds(0, 16)] = ibs[i][4 * t + v - 1, pl.ds(4, 16)]
                    rs[0, pl.ds(16, 16)] = ibs[i][4 * t + v, pl.ds(0, 16)]
                    ils[i][pl.ds(80 * t + 16 * v, 16)] = rs[0, pl.ds(kk, 16)]
                ils[i][pl.ds(80 * t + 64, 16)] = ibs[i][4 * t + 3, pl.ds(4, 16)]

        def fire(i, kk):
            pltpu.async_copy(
                table_hbm.at[ils[i].at[pl.ds(kk * CHUNK, CHUNK)]],
                rows[kk], sems[kk],
            )

        def drain(kk):
            pltpu.make_async_copy(
                table_hbm.at[ils[0].at[pl.ds(0, CHUNK)]], rows[kk], sems[kk]
            ).wait()

        def reduce_chunk(rref, acc):
            def body(r, acc):
                for u in range(UNROLL):
                    acc = tuple(
                        jnp.maximum(acc[j], rref[r * UNROLL + u, pl.ds(j * 16, 16)])
                        for j in range(NVREG)
                    )
                return acc
            return lax.fori_loop(0, CHUNK // UNROLL, body, acc)

        stage(0, 0)
        stage_wait(0)
        flatten(0)
        for kk in range(CPB):
            fire(0, kk)
        stage(1, 1)
        stage_wait(1)
        flatten(1)
        stage(2, 0)

        def batch_body(b, i, ni):
            acc = tuple(
                jnp.full((16,), -jnp.inf, jnp.float32) for _ in range(NVREG)
            )
            for kk in range(CPB):
                drain(kk)
                acc = reduce_chunk(rows[kk], acc)
                @pl.when(b + 1 < BPW)
                def _():
                    fire(ni, kk)
            for j in range(NVREG):
                out_v[b, pl.ds(j * 16, 16)] = acc[j]
            @pl.when(b + 3 < BPW)
            def _():
                stage(b + 3, ni)
            @pl.when(b + 2 < BPW)
            def _():
                stage_wait(i)
            flatten(i)

        def pair_body(p, _):
            batch_body(2 * p, 0, 1)
            batch_body(2 * p + 1, 1, 0)
            return 0

        lax.fori_loop(0, BPW // 2, pair_body, 0)
        pltpu.sync_copy(out_v, out_hbm.at[pl.ds(base, BPW)])

    return k(x, table)


def _fc_sigmoid(h, W, b2):
    def fc_kernel(h_ref, w_ref, b_ref, o_ref):
        acc = lax.dot_general(
            h_ref[...], w_ref[...],
            dimension_numbers=(((1,), (1,)), ((), ())),
            preferred_element_type=jnp.float32,
        )
        o_ref[...] = jax.nn.sigmoid(acc + b_ref[...])

    return pl.pallas_call(
        fc_kernel,
        out_shape=jax.ShapeDtypeStruct((B, NCLASS), jnp.float32),
    )(h, W, b2)


def kernel(x, table, W, b):
    h = _sc_gather_maxpool(x.astype(jnp.int32), table)
    return _fc_sigmoid(h, W, b.reshape(1, NCLASS))


# speedup vs baseline: 1.1078x; 1.0009x over previous
import functools

import jax
import jax.numpy as jnp
from jax import lax
from jax.experimental import pallas as pl
from jax.experimental.pallas import tpu as pltpu
from jax.experimental.pallas import tpu_sc as plsc

B, S, L = 1024, 20, 20
VOCAB, D, NCLASS = 100000, 128, 100

NIDX = S * L
CHUNK = 80
CPB = NIDX // CHUNK
NW = 32
BPW = B // NW
NVREG = D // 16
UNROLL = 4
NGRP = S // 4


def _sc_gather_maxpool(x, table):
    mesh = plsc.VectorSubcoreMesh(core_axis_name="c", subcore_axis_name="s")

    @functools.partial(
        pl.kernel,
        mesh=mesh,
        out_type=jax.ShapeDtypeStruct((B, D), jnp.float32),
        scratch_types=[
            pltpu.VMEM((S, L), jnp.int32),
            pltpu.VMEM((S, L), jnp.int32),
            pltpu.VMEM((NIDX,), jnp.int32),
            pltpu.VMEM((NIDX,), jnp.int32),
        ] + [pltpu.VMEM((CHUNK, D), jnp.float32)] * 10 + [
            pltpu.VMEM((BPW, D), jnp.float32),
            pltpu.VMEM((1, 32), jnp.int32),
        ] + [pltpu.SemaphoreType.DMA] * 12,
    )
    def k(x_hbm, table_hbm, out_hbm, ib0, ib1, il0, il1, *rest):
        rows = rest[0:10]
        out_v, rs = rest[10], rest[11]
        isems = rest[12:14]
        sems = rest[14:24]
        wid = lax.axis_index("s") * 2 + lax.axis_index("c")
        base = wid * BPW

        ibs = (ib0, ib1)
        ils = (il0, il1)


        def stage(b, i):
            pltpu.async_copy(x_hbm.at[base + b], ibs[i], isems[i])

        def stage_wait(i):
            pltpu.make_async_copy(x_hbm.at[0], ibs[i], isems[i]).wait()

        def flatten(i):
            for t in range(NGRP):
                ils[i][pl.ds(80 * t, 16)] = ibs[i][4 * t, pl.ds(0, 16)]
                for v, kk in ((1, 12), (2, 8), (3, 4)):
                    rs[0, pl.ds(0, 16)] = ibs[i][4 * t + v - 1, pl.ds(4, 16)]
                    rs[0, pl.ds(16, 16)] = ibs[i][4 * t + v, pl.ds(0, 16)]
                    ils[i][pl.ds(80 * t + 16 * v, 16)] = rs[0, pl.ds(kk, 16)]
                ils[i][pl.ds(80 * t + 64, 16)] = ibs[i][4 * t + 3, pl.ds(4, 16)]

        def fire(i, kk, slot):
            pltpu.async_copy(
                table_hbm.at[ils[i].at[pl.ds(kk * CHUNK, CHUNK)]],
                rows[slot], sems[slot],
            )

        def drain(slot):
            pltpu.make_async_copy(
                table_hbm.at[ils[0].at[pl.ds(0, CHUNK)]], rows[slot], sems[slot]
            ).wait()

        def reduce_chunk(rref, acc):
            def body(r, acc):
                for u in range(UNROLL):
                    acc = tuple(
                        jnp.maximum(acc[j], rref[r * UNROLL + u, pl.ds(j * 16, 16)])
                        for j in range(NVREG)
                    )
                return acc
            return lax.fori_loop(0, CHUNK // UNROLL, body, acc)

        stage(0, 0)
        stage_wait(0)
        flatten(0)
        for kk in range(CPB):
            fire(0, kk, kk)
        stage(1, 1)
        stage_wait(1)
        flatten(1)
        stage(2, 0)

        def batch_body(b, i, ni, mybase, nbase):
            acc = tuple(
                jnp.full((16,), -jnp.inf, jnp.float32) for _ in range(NVREG)
            )
            for kk in range(CPB):
                drain(mybase + kk)
                @pl.when(b + 1 < BPW)
                def _():
                    fire(ni, kk, nbase + kk)
                acc = reduce_chunk(rows[mybase + kk], acc)
            for j in range(NVREG):
                out_v[b, pl.ds(j * 16, 16)] = acc[j]
            @pl.when(b + 3 < BPW)
            def _():
                stage(b + 3, ni)
            @pl.when(b + 2 < BPW)
            def _():
                stage_wait(i)
            flatten(i)

        def pair_body(p, _):
            batch_body(2 * p, 0, 1, 0, CPB)
            batch_body(2 * p + 1, 1, 0, CPB, 0)
            return 0

        lax.fori_loop(0, BPW // 2, pair_body, 0)
        pltpu.sync_copy(out_v, out_hbm.at[pl.ds(base, BPW)])

    return k(x, table)


def _fc_sigmoid(h, W, b2):
    def fc_kernel(h_ref, w_ref, b_ref, o_ref):
        acc = lax.dot_general(
            h_ref[...], w_ref[...],
            dimension_numbers=(((1,), (1,)), ((), ())),
            preferred_element_type=jnp.float32,
        )
        o_ref[...] = jax.nn.sigmoid(acc + b_ref[...])

    return pl.pallas_call(
        fc_kernel,
        out_shape=jax.ShapeDtypeStruct((B, NCLASS), jnp.float32),
    )(h, W, b2)


def kernel(x, table, W, b):
    h = _sc_gather_maxpool(x.astype(jnp.int32), table)
    return _fc_sigmoid(h, W, b.reshape(1, NCLASS))
